# baseline (speedup 1.0000x reference)
import jax
import jax.numpy as jnp
from jax.experimental import pallas as pl

_N, _C, _T, _V = 128, 64, 64, 25
_L, _S, _INTER = 3, 3, 16
_INV = (1.0 + 1e-5) ** -0.5
_NEG = -1e30


def _tc_body(x2_ref, x4_ref, wdt_ref, bd_ref, pacat_ref, wsub_ref,
             mavg_ref, wa_ref, wb_ref, bb_ref, se_ref, bout_ref,
             out_ref):
    bf = jnp.bfloat16
    xn = x2_ref[0]
    xdt = jax.lax.dot_general(
        xn.astype(bf), wdt_ref[...], (((0,), (0,)), ((), ())),
        preferred_element_type=jnp.float32)
    yall = jnp.maximum(xdt + bd_ref[...], 0.0).astype(bf)
    y3 = yall.reshape(_T, _V, _L * _INTER)
    y3t = jnp.transpose(y3, (0, 2, 1))

    zsum = [None, None, None]
    esum = None
    for i in range(_L):
        xdb = y3t[:, i * _INTER:(i + 1) * _INTER, :].reshape(
            _T * _INTER, _V)

        xbar = jnp.dot(mavg_ref[...], xdb,
                       preferred_element_type=jnp.float32)
        zpre = jnp.dot(xdb, pacat_ref[i],
                       preferred_element_type=jnp.float32)
        zpre = zpre.astype(bf)
        for j in range(_S):
            zj = zpre[:, j * _V:(j + 1) * _V]
            parts = []
            for tb in range(8):
                chunk = zj[tb * 128:(tb + 1) * 128, :]
                parts.append(jnp.dot(wsub_ref[i, j], chunk,
                                     preferred_element_type=jnp.float32))
            zja = jnp.concatenate(parts, axis=0)
            zsum[j] = zja if zsum[j] is None else zsum[j] + zja

        xx = jnp.sum(xbar * xbar, axis=0)
        gram = jax.lax.dot_general(xbar, xbar, (((0,), (0,)), ((), ())),
                                   preferred_element_type=jnp.float32)
        scoret = 2.0 * gram - xx[:, None]
        s = scoret
        for _ in range(4):
            m = jnp.max(s, axis=0, keepdims=True)
            s = jnp.where(s == m, _NEG, s)
        thresh = jnp.max(s, axis=0, keepdims=True)
        maskt = scoret >= thresh
        a2 = jnp.dot(wa_ref[i], xbar,
                     preferred_element_type=jnp.float32)
        b2 = jnp.dot(wb_ref[i], xbar,
                     preferred_element_type=jnp.float32)
        b2 = b2 + bb_ref[i]
        amax = None
        for u in range(_V):
            cand = jnp.where(maskt[u:u + 1, :], a2[:, u:u + 1], _NEG)
            amax = cand if amax is None else jnp.maximum(amax, cand)
        e2 = amax + b2
        e2 = jnp.where(e2 > 0, e2, 0.2 * e2)
        esum = e2 if esum is None else esum + e2

    es = esum * se_ref[...]
    ebc = jnp.broadcast_to(es[:, None, :], (_INTER, _T, _V))
    zall = jnp.concatenate(
        [zsum[j].reshape(_T, _INTER, _V) for j in range(_S)],
        axis=1)
    zallt = jnp.transpose(zall, (1, 0, 2))
    zcat3 = jnp.concatenate([zallt, ebc], axis=0)
    out3 = zcat3 + bout_ref[...] + x4_ref[0]
    out_ref[0] = jnp.maximum(out3, 0.0)


def _run(x2, x4, Wdt, bd, PAcat, Wsub, Mavg, WA, WB, bb, se, bout,
         interpret=False):
    return pl.pallas_call(
        _tc_body,
        grid=(_N,),
        in_specs=[
            pl.BlockSpec((1, _C, _T * _V), lambda n: (n, 0, 0)),
            pl.BlockSpec((1, _C, _T, _V), lambda n: (n, 0, 0, 0)),
            pl.BlockSpec((_C, _L * _INTER), lambda n: (0, 0)),
            pl.BlockSpec((1, _L * _INTER), lambda n: (0, 0)),
            pl.BlockSpec((_L, _V, _S * _V), lambda n: (0, 0, 0)),
            pl.BlockSpec((_L, _S, 128, 128), lambda n: (0, 0, 0, 0)),
            pl.BlockSpec((_INTER, _T * _INTER), lambda n: (0, 0)),
            pl.BlockSpec((_L, _INTER, _INTER), lambda n: (0, 0, 0)),
            pl.BlockSpec((_L, _INTER, _INTER), lambda n: (0, 0, 0)),
            pl.BlockSpec((_L, _INTER, 1), lambda n: (0, 0, 0)),
            pl.BlockSpec((_INTER, 1), lambda n: (0, 0)),
            pl.BlockSpec((_C, 1, 1), lambda n: (0, 0, 0)),
        ],
        out_specs=pl.BlockSpec((1, _C, _T, _V), lambda n: (n, 0, 0, 0)),
        out_shape=jax.ShapeDtypeStruct((_N, _C, _T, _V), jnp.float32),
        interpret=interpret,
    )(x2, x4, Wdt, bd, PAcat, Wsub, Mavg, WA, WB, bb, se, bout)


def _prep(x, PA, Wdown, bdown, gdown, betdown, Wsub, bsub, gsub, betsub,
          Wedge, gedge, betedge, gbn, bbn):
    x2 = x.reshape(_N, _C, _T * _V)
    sdown = gdown * _INV
    Wd = (Wdown * sdown[:, :, None]).reshape(_L * _INTER, _C)
    Wdt = Wd.T
    bd = (bdown * sdown + betdown).reshape(1, _L * _INTER)
    PAcat = jnp.transpose(PA, (0, 3, 1, 2)).reshape(_L, _V, _S * _V)
    ssub = gsub * _INV
    Wsubf = Wsub * ssub[..., None]
    szr = (gbn * _INV)[:_S * _INTER].reshape(_S, _INTER)
    Wsubg = Wsubf * szr[None, :, :, None]
    eye8 = jnp.eye(8, dtype=x.dtype)
    Wblk = jnp.einsum('tu,ijoc->ijtouc', eye8, Wsubg).reshape(
        _L, _S, 128, 128)
    Mavg = jnp.tile(jnp.eye(_INTER, dtype=x.dtype) / _T, (1, _T))
    sedge = gedge * _INV
    W1 = Wedge[:, :, :_INTER]
    W2 = Wedge[:, :, _INTER:]
    WA = W1 * sedge[..., None]
    WB = (W2 - W1) * sedge[..., None]
    bb = betedge.reshape(_L, _INTER, 1)
    se = (gbn * _INV)[_S * _INTER:].reshape(_INTER, 1)
    bsubf = (bsub * ssub + betsub)
    bias_z = jnp.sum(bsubf, axis=0)
    bias_full = jnp.concatenate(
        [bias_z.reshape(_S * _INTER), jnp.zeros((_INTER,), x.dtype)])
    bout = (bias_full * (gbn * _INV) + bbn).reshape(_C, 1, 1)
    return x2, x, Wdt, bd, PAcat, Wblk, Mavg, WA, WB, bb, se, bout


@jax.jit
def kernel(x, PA, Wdown, bdown, gdown, betdown, Wsub, bsub, gsub, betsub,
           Wedge, gedge, betedge, gbn, bbn):
    args = _prep(x, PA, Wdown, bdown, gdown, betdown, Wsub, bsub, gsub,
                 betsub, Wedge, gedge, betedge, gbn, bbn)
    return _run(*args)


# speedup vs baseline: 1.2343x; 1.2343x over previous
import jax
import jax.numpy as jnp
from jax.experimental import pallas as pl

_N, _C, _T, _V = 128, 64, 64, 25
_L, _S, _INTER = 3, 3, 16
_INV = (1.0 + 1e-5) ** -0.5
_NEG = -1e30


def _tc_body(x2_ref, x4_ref, wdt_ref, bd_ref, pacat_ref, wsub_ref,
             mavg_ref, wa_ref, wb_ref, bb_ref, se_ref, bout_ref,
             out_ref):
    bf = jnp.bfloat16
    xn = x2_ref[0]
    xdt = jax.lax.dot_general(
        xn.astype(bf), wdt_ref[...], (((0,), (0,)), ((), ())),
        preferred_element_type=jnp.float32)
    yall = jnp.maximum(xdt + bd_ref[...], 0.0).astype(bf)
    y3 = yall.reshape(_T, _V, _L * _INTER)
    y3t = jnp.transpose(y3, (0, 2, 1))

    zsum = [None, None, None]
    esum = None
    for i in range(_L):
        xdb = y3t[:, i * _INTER:(i + 1) * _INTER, :].reshape(
            _T * _INTER, _V)

        xbar = jnp.dot(mavg_ref[...], xdb,
                       preferred_element_type=jnp.float32)
        zpre = jnp.dot(xdb, pacat_ref[i],
                       preferred_element_type=jnp.float32)
        zpre = zpre.astype(bf)
        for j in range(_S):
            zj = zpre[:, j * _V:(j + 1) * _V]
            parts = []
            for tb in range(8):
                chunk = zj[tb * 128:(tb + 1) * 128, :]
                parts.append(jnp.dot(wsub_ref[i, j], chunk,
                                     preferred_element_type=jnp.float32))
            zja = jnp.concatenate(parts, axis=0)
            zsum[j] = zja if zsum[j] is None else zsum[j] + zja

        xx = jnp.sum(xbar * xbar, axis=0)
        gram = jax.lax.dot_general(xbar, xbar, (((0,), (0,)), ((), ())),
                                   preferred_element_type=jnp.float32)
        scoret = 2.0 * gram - xx[:, None]
        s = scoret
        for _ in range(4):
            m = jnp.max(s, axis=0, keepdims=True)
            s = jnp.where(s == m, _NEG, s)
        thresh = jnp.max(s, axis=0, keepdims=True)
        maskt = scoret >= thresh
        a2 = jnp.dot(wa_ref[i], xbar,
                     preferred_element_type=jnp.float32)
        b2 = jnp.dot(wb_ref[i], xbar,
                     preferred_element_type=jnp.float32)
        b2 = b2 + bb_ref[i]
        amax = None
        for u in range(_V):
            cand = jnp.where(maskt[u:u + 1, :], a2[:, u:u + 1], _NEG)
            amax = cand if amax is None else jnp.maximum(amax, cand)
        e2 = amax + b2
        e2 = jnp.where(e2 > 0, e2, 0.2 * e2)
        esum = e2 if esum is None else esum + e2

    es = esum * se_ref[...]
    ebc = jnp.broadcast_to(es[:, None, :], (_INTER, _T, _V))
    zall = jnp.concatenate(
        [zsum[j].reshape(_T, _INTER, _V) for j in range(_S)],
        axis=1)
    zallt = jnp.transpose(zall, (1, 0, 2))
    zcat3 = jnp.concatenate([zallt, ebc], axis=0)
    xres = x4_ref[0].reshape(_C, _T, _V)
    out3 = zcat3 + bout_ref[...] + xres
    out_ref[0] = jnp.maximum(out3, 0.0).reshape(_C * _T, _V)


def _run(x2, x4, Wdt, bd, PAcat, Wsub, Mavg, WA, WB, bb, se, bout,
         interpret=False):
    return pl.pallas_call(
        _tc_body,
        grid=(_N,),
        in_specs=[
            pl.BlockSpec((1, _C, _T * _V), lambda n: (n, 0, 0)),
            pl.BlockSpec((1, _C * _T, _V), lambda n: (n, 0, 0)),
            pl.BlockSpec((_C, _L * _INTER), lambda n: (0, 0)),
            pl.BlockSpec((1, _L * _INTER), lambda n: (0, 0)),
            pl.BlockSpec((_L, _V, _S * _V), lambda n: (0, 0, 0)),
            pl.BlockSpec((_L, _S, 128, 128), lambda n: (0, 0, 0, 0)),
            pl.BlockSpec((_INTER, _T * _INTER), lambda n: (0, 0)),
            pl.BlockSpec((_L, _INTER, _INTER), lambda n: (0, 0, 0)),
            pl.BlockSpec((_L, _INTER, _INTER), lambda n: (0, 0, 0)),
            pl.BlockSpec((_L, _INTER, 1), lambda n: (0, 0, 0)),
            pl.BlockSpec((_INTER, 1), lambda n: (0, 0)),
            pl.BlockSpec((_C, 1, 1), lambda n: (0, 0, 0)),
        ],
        out_specs=pl.BlockSpec((1, _C * _T, _V), lambda n: (n, 0, 0)),
        out_shape=jax.ShapeDtypeStruct((_N, _C * _T, _V), jnp.float32),
        interpret=interpret,
    )(x2, x4, Wdt, bd, PAcat, Wsub, Mavg, WA, WB, bb, se, bout)


def _prep(x, PA, Wdown, bdown, gdown, betdown, Wsub, bsub, gsub, betsub,
          Wedge, gedge, betedge, gbn, bbn):
    x2 = x.reshape(_N, _C, _T * _V)
    x3 = x.reshape(_N, _C * _T, _V)
    sdown = gdown * _INV
    Wd = (Wdown * sdown[:, :, None]).reshape(_L * _INTER, _C)
    Wdt = Wd.T
    bd = (bdown * sdown + betdown).reshape(1, _L * _INTER)
    PAcat = jnp.transpose(PA, (0, 3, 1, 2)).reshape(_L, _V, _S * _V)
    ssub = gsub * _INV
    Wsubf = Wsub * ssub[..., None]
    szr = (gbn * _INV)[:_S * _INTER].reshape(_S, _INTER)
    Wsubg = Wsubf * szr[None, :, :, None]
    eye8 = jnp.eye(8, dtype=x.dtype)
    Wblk = jnp.einsum('tu,ijoc->ijtouc', eye8, Wsubg).reshape(
        _L, _S, 128, 128)
    Mavg = jnp.tile(jnp.eye(_INTER, dtype=x.dtype) / _T, (1, _T))
    sedge = gedge * _INV
    W1 = Wedge[:, :, :_INTER]
    W2 = Wedge[:, :, _INTER:]
    WA = W1 * sedge[..., None]
    WB = (W2 - W1) * sedge[..., None]
    bb = betedge.reshape(_L, _INTER, 1)
    se = (gbn * _INV)[_S * _INTER:].reshape(_INTER, 1)
    bsubf = (bsub * ssub + betsub)
    bias_z = jnp.sum(bsubf, axis=0)
    bias_full = jnp.concatenate(
        [bias_z.reshape(_S * _INTER), jnp.zeros((_INTER,), x.dtype)])
    bout = (bias_full * (gbn * _INV) + bbn).reshape(_C, 1, 1)
    return x2, x3, Wdt, bd, PAcat, Wblk, Mavg, WA, WB, bb, se, bout


@jax.jit
def kernel(x, PA, Wdown, bdown, gdown, betdown, Wsub, bsub, gsub, betsub,
           Wedge, gedge, betedge, gbn, bbn):
    args = _prep(x, PA, Wdown, bdown, gdown, betdown, Wsub, bsub, gsub,
                 betsub, Wedge, gedge, betedge, gbn, bbn)
    return _run(*args).reshape(_N, _C, _T, _V)


# speedup vs baseline: 1.2715x; 1.0301x over previous
import jax
import jax.numpy as jnp
from jax.experimental import pallas as pl
from jax.experimental.pallas import tpu as pltpu

_N, _C, _T, _V = 128, 64, 64, 25
_L, _S, _INTER = 3, 3, 16
_B = 2
_INV = (1.0 + 1e-5) ** -0.5
_NEG = -1e30


def _tc_body(x2_ref, x4_ref, wdt_ref, bd_ref, pacat_ref, wsub_ref,
             mavg_ref, wa_ref, wb_ref, bb_ref, se_ref, bout_ref,
             out_ref):
    bf = jnp.bfloat16
    for smp in range(_B):
        _one_sample(smp, x2_ref, x4_ref, wdt_ref, bd_ref, pacat_ref,
                    wsub_ref, mavg_ref, wa_ref, wb_ref, bb_ref, se_ref,
                    bout_ref, out_ref)


def _one_sample(smp, x2_ref, x4_ref, wdt_ref, bd_ref, pacat_ref, wsub_ref,
                mavg_ref, wa_ref, wb_ref, bb_ref, se_ref, bout_ref,
                out_ref):
    bf = jnp.bfloat16
    xn = x2_ref[smp]
    xdt = jnp.dot(xn.astype(bf).T, wdt_ref[...],
                  preferred_element_type=jnp.float32)
    yall = jnp.maximum(xdt + bd_ref[...], 0.0).astype(bf)
    y3 = yall.reshape(_T, _V, _L * _INTER)
    y3t = jnp.transpose(y3, (0, 2, 1))

    zsum = [None, None, None]
    esum = None
    for i in range(_L):
        xdb = y3t[:, i * _INTER:(i + 1) * _INTER, :].reshape(
            _T * _INTER, _V)

        xbar = jnp.dot(mavg_ref[...], xdb,
                       preferred_element_type=jnp.float32)
        zpre = jnp.dot(xdb, pacat_ref[i],
                       preferred_element_type=jnp.float32)
        for j in range(_S):
            zj = zpre[:, j * _V:(j + 1) * _V]
            parts = []
            for tb in range(8):
                chunk = zj[tb * 128:(tb + 1) * 128, :]
                parts.append(jnp.dot(wsub_ref[i, j], chunk,
                                     preferred_element_type=jnp.float32))
            zja = jnp.concatenate(parts, axis=0)
            zsum[j] = zja if zsum[j] is None else zsum[j] + zja

        xx = jnp.sum(xbar * xbar, axis=0)
        gram = jax.lax.dot_general(xbar, xbar, (((0,), (0,)), ((), ())),
                                   preferred_element_type=jnp.float32)
        scoret = 2.0 * gram - xx[:, None]
        s = scoret
        for _ in range(4):
            m = jnp.max(s, axis=0, keepdims=True)
            s = jnp.where(s == m, _NEG, s)
        thresh = jnp.max(s, axis=0, keepdims=True)
        maskt = scoret >= thresh
        a2 = jnp.dot(wa_ref[i], xbar,
                     preferred_element_type=jnp.float32)
        b2 = jnp.dot(wb_ref[i], xbar,
                     preferred_element_type=jnp.float32)
        b2 = b2 + bb_ref[i]
        amax = None
        for u in range(_V):
            cand = jnp.where(maskt[u:u + 1, :], a2[:, u:u + 1], _NEG)
            amax = cand if amax is None else jnp.maximum(amax, cand)
        e2 = amax + b2
        e2 = jnp.where(e2 > 0, e2, 0.2 * e2)
        esum = e2 if esum is None else esum + e2

    es = esum * se_ref[...]
    ebc = jnp.broadcast_to(es[:, None, :], (_INTER, _T, _V))
    zall = jnp.concatenate(
        [zsum[j].reshape(_T, _INTER, _V) for j in range(_S)],
        axis=1)
    zallt = jnp.transpose(zall, (1, 0, 2))
    zcat3 = jnp.concatenate([zallt, ebc], axis=0)
    xres = x4_ref[smp].reshape(_C, _T, _V)
    out3 = zcat3 + bout_ref[...] + xres
    out_ref[smp] = jnp.maximum(out3, 0.0).reshape(_C * _T, _V)


def _run(x2, x4, Wdt, bd, PAcat, Wsub, Mavg, WA, WB, bb, se, bout,
         interpret=False):
    return pl.pallas_call(
        _tc_body,
        grid=(_N // _B,),
        in_specs=[
            pl.BlockSpec((_B, _C, _T * _V), lambda n: (n, 0, 0)),
            pl.BlockSpec((_B, _C * _T, _V), lambda n: (n, 0, 0)),
            pl.BlockSpec((_C, _L * _INTER), lambda n: (0, 0)),
            pl.BlockSpec((1, _L * _INTER), lambda n: (0, 0)),
            pl.BlockSpec((_L, _V, _S * _V), lambda n: (0, 0, 0)),
            pl.BlockSpec((_L, _S, 128, 128), lambda n: (0, 0, 0, 0)),
            pl.BlockSpec((_INTER, _T * _INTER), lambda n: (0, 0)),
            pl.BlockSpec((_L, _INTER, _INTER), lambda n: (0, 0, 0)),
            pl.BlockSpec((_L, _INTER, _INTER), lambda n: (0, 0, 0)),
            pl.BlockSpec((_L, _INTER, 1), lambda n: (0, 0, 0)),
            pl.BlockSpec((_INTER, 1), lambda n: (0, 0)),
            pl.BlockSpec((_C, 1, 1), lambda n: (0, 0, 0)),
        ],
        out_specs=pl.BlockSpec((_B, _C * _T, _V), lambda n: (n, 0, 0)),
        out_shape=jax.ShapeDtypeStruct((_N, _C * _T, _V), jnp.float32),
        compiler_params=pltpu.CompilerParams(
            fuse_transposed_lhs_in_matmul=True),
        interpret=interpret,
    )(x2, x4, Wdt, bd, PAcat, Wsub, Mavg, WA, WB, bb, se, bout)


def _prep(x, PA, Wdown, bdown, gdown, betdown, Wsub, bsub, gsub, betsub,
          Wedge, gedge, betedge, gbn, bbn):
    x2 = x.reshape(_N, _C, _T * _V)
    x3 = x.reshape(_N, _C * _T, _V)
    sdown = gdown * _INV
    Wd = (Wdown * sdown[:, :, None]).reshape(_L * _INTER, _C)
    Wdt = Wd.T
    bd = (bdown * sdown + betdown).reshape(1, _L * _INTER)
    PAcat = jnp.transpose(PA, (0, 3, 1, 2)).reshape(_L, _V, _S * _V)
    ssub = gsub * _INV
    Wsubf = Wsub * ssub[..., None]
    szr = (gbn * _INV)[:_S * _INTER].reshape(_S, _INTER)
    Wsubg = Wsubf * szr[None, :, :, None]
    eye8 = jnp.eye(8, dtype=x.dtype)
    Wblk = jnp.einsum('tu,ijoc->ijtouc', eye8, Wsubg).reshape(
        _L, _S, 128, 128)
    Mavg = jnp.tile(jnp.eye(_INTER, dtype=x.dtype) / _T, (1, _T))
    sedge = gedge * _INV
    W1 = Wedge[:, :, :_INTER]
    W2 = Wedge[:, :, _INTER:]
    WA = W1 * sedge[..., None]
    WB = (W2 - W1) * sedge[..., None]
    bb = betedge.reshape(_L, _INTER, 1)
    se = (gbn * _INV)[_S * _INTER:].reshape(_INTER, 1)
    bsubf = (bsub * ssub + betsub)
    bias_z = jnp.sum(bsubf, axis=0)
    bias_full = jnp.concatenate(
        [bias_z.reshape(_S * _INTER), jnp.zeros((_INTER,), x.dtype)])
    bout = (bias_full * (gbn * _INV) + bbn).reshape(_C, 1, 1)
    return x2, x3, Wdt, bd, PAcat, Wblk, Mavg, WA, WB, bb, se, bout


@jax.jit
def kernel(x, PA, Wdown, bdown, gdown, betdown, Wsub, bsub, gsub, betsub,
           Wedge, gedge, betedge, gbn, bbn):
    args = _prep(x, PA, Wdown, bdown, gdown, betdown, Wsub, bsub, gsub,
                 betsub, Wedge, gedge, betedge, gbn, bbn)
    return _run(*args).reshape(_N, _C, _T, _V)


# speedup vs baseline: 1.2803x; 1.0069x over previous
import jax
import jax.numpy as jnp
from jax.experimental import pallas as pl
from jax.experimental.pallas import tpu as pltpu

_N, _C, _T, _V = 128, 64, 64, 25
_L, _S, _INTER = 3, 3, 16
_B = 4
_INV = (1.0 + 1e-5) ** -0.5
_NEG = -1e30


def _tc_body(x2_ref, x4_ref, wdt_ref, bd_ref, pacat_ref, wsub_ref,
             mavg_ref, wa_ref, wb_ref, bb_ref, se_ref, bout_ref,
             out_ref):
    bf = jnp.bfloat16
    for smp in range(_B):
        _one_sample(smp, x2_ref, x4_ref, wdt_ref, bd_ref, pacat_ref,
                    wsub_ref, mavg_ref, wa_ref, wb_ref, bb_ref, se_ref,
                    bout_ref, out_ref)


def _one_sample(smp, x2_ref, x4_ref, wdt_ref, bd_ref, pacat_ref, wsub_ref,
                mavg_ref, wa_ref, wb_ref, bb_ref, se_ref, bout_ref,
                out_ref):
    bf = jnp.bfloat16
    xn = x2_ref[smp]
    xdt = jnp.dot(xn.astype(bf).T, wdt_ref[...],
                  preferred_element_type=jnp.float32)
    yall = jnp.maximum(xdt + bd_ref[...], 0.0).astype(bf)
    y3 = yall.reshape(_T, _V, _L * _INTER)
    y3t = jnp.transpose(y3, (0, 2, 1))

    zsum = [None, None, None]
    esum = None
    for i in range(_L):
        xdb = y3t[:, i * _INTER:(i + 1) * _INTER, :].reshape(
            _T * _INTER, _V)

        xbar = jnp.dot(mavg_ref[...], xdb,
                       preferred_element_type=jnp.float32)
        zpre = jnp.dot(xdb, pacat_ref[i],
                       preferred_element_type=jnp.float32)
        for j in range(_S):
            zj = zpre[:, j * _V:(j + 1) * _V]
            parts = []
            for tb in range(8):
                chunk = zj[tb * 128:(tb + 1) * 128, :]
                parts.append(jnp.dot(wsub_ref[i, j], chunk,
                                     preferred_element_type=jnp.float32))
            zja = jnp.concatenate(parts, axis=0)
            zsum[j] = zja if zsum[j] is None else zsum[j] + zja

        xx = jnp.sum(xbar * xbar, axis=0)
        gram = jax.lax.dot_general(xbar, xbar, (((0,), (0,)), ((), ())),
                                   preferred_element_type=jnp.float32)
        scoret = 2.0 * gram - xx[:, None]
        s = scoret
        for _ in range(4):
            m = jnp.max(s, axis=0, keepdims=True)
            s = jnp.where(s == m, _NEG, s)
        thresh = jnp.max(s, axis=0, keepdims=True)
        maskt = scoret >= thresh
        a2 = jnp.dot(wa_ref[i], xbar,
                     preferred_element_type=jnp.float32)
        b2 = jnp.dot(wb_ref[i], xbar,
                     preferred_element_type=jnp.float32)
        b2 = b2 + bb_ref[i]
        amax = None
        for u in range(_V):
            cand = jnp.where(maskt[u:u + 1, :], a2[:, u:u + 1], _NEG)
            amax = cand if amax is None else jnp.maximum(amax, cand)
        e2 = amax + b2
        e2 = jnp.where(e2 > 0, e2, 0.2 * e2)
        esum = e2 if esum is None else esum + e2

    es = esum * se_ref[...]
    ebc = jnp.broadcast_to(es[:, None, :], (_INTER, _T, _V))
    zall = jnp.concatenate(
        [zsum[j].reshape(_T, _INTER, _V) for j in range(_S)],
        axis=1)
    zallt = jnp.transpose(zall, (1, 0, 2))
    zcat3 = jnp.concatenate([zallt, ebc], axis=0)
    xres = x4_ref[smp].reshape(_C, _T, _V)
    out3 = zcat3 + bout_ref[...] + xres
    out_ref[smp] = jnp.maximum(out3, 0.0).reshape(_C * _T, _V)


def _run(x2, x4, Wdt, bd, PAcat, Wsub, Mavg, WA, WB, bb, se, bout,
         interpret=False):
    return pl.pallas_call(
        _tc_body,
        grid=(_N // _B,),
        in_specs=[
            pl.BlockSpec((_B, _C, _T * _V), lambda n: (n, 0, 0)),
            pl.BlockSpec((_B, _C * _T, _V), lambda n: (n, 0, 0)),
            pl.BlockSpec((_C, _L * _INTER), lambda n: (0, 0)),
            pl.BlockSpec((1, _L * _INTER), lambda n: (0, 0)),
            pl.BlockSpec((_L, _V, _S * _V), lambda n: (0, 0, 0)),
            pl.BlockSpec((_L, _S, 128, 128), lambda n: (0, 0, 0, 0)),
            pl.BlockSpec((_INTER, _T * _INTER), lambda n: (0, 0)),
            pl.BlockSpec((_L, _INTER, _INTER), lambda n: (0, 0, 0)),
            pl.BlockSpec((_L, _INTER, _INTER), lambda n: (0, 0, 0)),
            pl.BlockSpec((_L, _INTER, 1), lambda n: (0, 0, 0)),
            pl.BlockSpec((_INTER, 1), lambda n: (0, 0)),
            pl.BlockSpec((_C, 1, 1), lambda n: (0, 0, 0)),
        ],
        out_specs=pl.BlockSpec((_B, _C * _T, _V), lambda n: (n, 0, 0)),
        out_shape=jax.ShapeDtypeStruct((_N, _C * _T, _V), jnp.float32),
        compiler_params=pltpu.CompilerParams(
            fuse_transposed_lhs_in_matmul=True),
        interpret=interpret,
    )(x2, x4, Wdt, bd, PAcat, Wsub, Mavg, WA, WB, bb, se, bout)


def _prep(x, PA, Wdown, bdown, gdown, betdown, Wsub, bsub, gsub, betsub,
          Wedge, gedge, betedge, gbn, bbn):
    x2 = x.reshape(_N, _C, _T * _V)
    x3 = x.reshape(_N, _C * _T, _V)
    sdown = gdown * _INV
    Wd = (Wdown * sdown[:, :, None]).reshape(_L * _INTER, _C)
    Wdt = Wd.T
    bd = (bdown * sdown + betdown).reshape(1, _L * _INTER)
    PAcat = jnp.transpose(PA, (0, 3, 1, 2)).reshape(_L, _V, _S * _V)
    ssub = gsub * _INV
    Wsubf = Wsub * ssub[..., None]
    szr = (gbn * _INV)[:_S * _INTER].reshape(_S, _INTER)
    Wsubg = Wsubf * szr[None, :, :, None]
    eye8 = jnp.eye(8, dtype=x.dtype)
    Wblk = jnp.einsum('tu,ijoc->ijtouc', eye8, Wsubg).reshape(
        _L, _S, 128, 128)
    Mavg = jnp.tile(jnp.eye(_INTER, dtype=x.dtype) / _T, (1, _T))
    sedge = gedge * _INV
    W1 = Wedge[:, :, :_INTER]
    W2 = Wedge[:, :, _INTER:]
    WA = W1 * sedge[..., None]
    WB = (W2 - W1) * sedge[..., None]
    bb = betedge.reshape(_L, _INTER, 1)
    se = (gbn * _INV)[_S * _INTER:].reshape(_INTER, 1)
    bsubf = (bsub * ssub + betsub)
    bias_z = jnp.sum(bsubf, axis=0)
    bias_full = jnp.concatenate(
        [bias_z.reshape(_S * _INTER), jnp.zeros((_INTER,), x.dtype)])
    bout = (bias_full * (gbn * _INV) + bbn).reshape(_C, 1, 1)
    return x2, x3, Wdt, bd, PAcat, Wblk, Mavg, WA, WB, bb, se, bout


@jax.jit
def kernel(x, PA, Wdown, bdown, gdown, betdown, Wsub, bsub, gsub, betsub,
           Wedge, gedge, betedge, gbn, bbn):
    args = _prep(x, PA, Wdown, bdown, gdown, betdown, Wsub, bsub, gsub,
                 betsub, Wedge, gedge, betedge, gbn, bbn)
    return _run(*args).reshape(_N, _C, _T, _V)


# speedup vs baseline: 1.3402x; 1.0468x over previous
import jax
import jax.numpy as jnp
from jax.experimental import pallas as pl
from jax.experimental.pallas import tpu as pltpu

_N, _C, _T, _V = 128, 64, 64, 25
_L, _S, _INTER = 3, 3, 16
_B = 4
_INV = (1.0 + 1e-5) ** -0.5
_NEG = -1e30


def _tc_body(x2_ref, x4_ref, wdt_ref, bd_ref, pacat_ref, wsub_ref,
             mavg_ref, wa_ref, wb_ref, bb_ref, se_ref, bout_ref,
             out_ref):
    bf = jnp.bfloat16
    for smp in range(_B):
        _one_sample(smp, x2_ref, x4_ref, wdt_ref, bd_ref, pacat_ref,
                    wsub_ref, mavg_ref, wa_ref, wb_ref, bb_ref, se_ref,
                    bout_ref, out_ref)


def _one_sample(smp, x2_ref, x4_ref, wdt_ref, bd_ref, pacat_ref, wsub_ref,
                mavg_ref, wa_ref, wb_ref, bb_ref, se_ref, bout_ref,
                out_ref):
    bf = jnp.bfloat16
    xn = x2_ref[smp]
    xdt = jnp.dot(xn.astype(bf).T, wdt_ref[...],
                  preferred_element_type=jnp.float32)
    yall = jnp.maximum(xdt + bd_ref[...], 0.0).astype(bf)
    y3 = yall.reshape(_T, _V, _L * _INTER)
    y3t = jnp.transpose(y3, (0, 2, 1))

    zsum = [None, None, None]
    esum = None
    for i in range(_L):
        xdb = y3t[:, i * _INTER:(i + 1) * _INTER, :].reshape(
            _T * _INTER, _V)

        xbar = jnp.dot(mavg_ref[...], xdb,
                       preferred_element_type=jnp.float32)
        zpre = jnp.dot(xdb, pacat_ref[i],
                       preferred_element_type=jnp.float32)
        for j in range(_S):
            zj = zpre[:, j * _V:(j + 1) * _V]
            parts = []
            for tb in range(8):
                chunk = zj[tb * 128:(tb + 1) * 128, :]
                parts.append(jnp.dot(wsub_ref[i, j], chunk,
                                     preferred_element_type=jnp.float32)
                             .reshape(_INTER, 1, 8, _V))
            zja = jnp.concatenate(parts, axis=1).reshape(
                _INTER * _T, _V)
            zsum[j] = zja if zsum[j] is None else zsum[j] + zja

        xx = jnp.sum(xbar * xbar, axis=0)
        gram = jax.lax.dot_general(xbar, xbar, (((0,), (0,)), ((), ())),
                                   preferred_element_type=jnp.float32)
        scoret = 2.0 * gram - xx[:, None]
        s = scoret
        for _ in range(4):
            m = jnp.max(s, axis=0, keepdims=True)
            s = jnp.where(s == m, _NEG, s)
        thresh = jnp.max(s, axis=0, keepdims=True)
        maskt = scoret >= thresh
        a2 = jnp.dot(wa_ref[i], xbar,
                     preferred_element_type=jnp.float32)
        b2 = jnp.dot(wb_ref[i], xbar,
                     preferred_element_type=jnp.float32)
        b2 = b2 + bb_ref[i]
        amax = None
        for u in range(_V):
            cand = jnp.where(maskt[u:u + 1, :], a2[:, u:u + 1], _NEG)
            amax = cand if amax is None else jnp.maximum(amax, cand)
        e2 = amax + b2
        e2 = jnp.where(e2 > 0, e2, 0.2 * e2)
        esum = e2 if esum is None else esum + e2

    es = esum * se_ref[...]
    ebc = jnp.broadcast_to(es[:, None, :], (_INTER, _T, _V))
    zcat3 = jnp.concatenate(
        [zsum[j].reshape(_INTER, _T, _V) for j in range(_S)] + [ebc],
        axis=0)
    xres = x4_ref[smp].reshape(_C, _T, _V)
    out3 = zcat3 + bout_ref[...] + xres
    out_ref[smp] = jnp.maximum(out3, 0.0).reshape(_C * _T, _V)


def _run(x2, x4, Wdt, bd, PAcat, Wsub, Mavg, WA, WB, bb, se, bout,
         interpret=False):
    return pl.pallas_call(
        _tc_body,
        grid=(_N // _B,),
        in_specs=[
            pl.BlockSpec((_B, _C, _T * _V), lambda n: (n, 0, 0)),
            pl.BlockSpec((_B, _C * _T, _V), lambda n: (n, 0, 0)),
            pl.BlockSpec((_C, _L * _INTER), lambda n: (0, 0)),
            pl.BlockSpec((1, _L * _INTER), lambda n: (0, 0)),
            pl.BlockSpec((_L, _V, _S * _V), lambda n: (0, 0, 0)),
            pl.BlockSpec((_L, _S, 128, 128), lambda n: (0, 0, 0, 0)),
            pl.BlockSpec((_INTER, _T * _INTER), lambda n: (0, 0)),
            pl.BlockSpec((_L, _INTER, _INTER), lambda n: (0, 0, 0)),
            pl.BlockSpec((_L, _INTER, _INTER), lambda n: (0, 0, 0)),
            pl.BlockSpec((_L, _INTER, 1), lambda n: (0, 0, 0)),
            pl.BlockSpec((_INTER, 1), lambda n: (0, 0)),
            pl.BlockSpec((_C, 1, 1), lambda n: (0, 0, 0)),
        ],
        out_specs=pl.BlockSpec((_B, _C * _T, _V), lambda n: (n, 0, 0)),
        out_shape=jax.ShapeDtypeStruct((_N, _C * _T, _V), jnp.float32),
        compiler_params=pltpu.CompilerParams(
            fuse_transposed_lhs_in_matmul=True),
        interpret=interpret,
    )(x2, x4, Wdt, bd, PAcat, Wsub, Mavg, WA, WB, bb, se, bout)


def _prep(x, PA, Wdown, bdown, gdown, betdown, Wsub, bsub, gsub, betsub,
          Wedge, gedge, betedge, gbn, bbn):
    x2 = x.reshape(_N, _C, _T * _V)
    x3 = x.reshape(_N, _C * _T, _V)
    sdown = gdown * _INV
    Wd = (Wdown * sdown[:, :, None]).reshape(_L * _INTER, _C)
    Wdt = Wd.T
    bd = (bdown * sdown + betdown).reshape(1, _L * _INTER)
    PAcat = jnp.transpose(PA, (0, 3, 1, 2)).reshape(_L, _V, _S * _V)
    ssub = gsub * _INV
    Wsubf = Wsub * ssub[..., None]
    szr = (gbn * _INV)[:_S * _INTER].reshape(_S, _INTER)
    Wsubg = Wsubf * szr[None, :, :, None]
    eye8 = jnp.eye(8, dtype=x.dtype)
    Wblk = jnp.einsum('tu,ijoc->ijotuc', eye8, Wsubg).reshape(
        _L, _S, 128, 128)
    Mavg = jnp.tile(jnp.eye(_INTER, dtype=x.dtype) / _T, (1, _T))
    sedge = gedge * _INV
    W1 = Wedge[:, :, :_INTER]
    W2 = Wedge[:, :, _INTER:]
    WA = W1 * sedge[..., None]
    WB = (W2 - W1) * sedge[..., None]
    bb = betedge.reshape(_L, _INTER, 1)
    se = (gbn * _INV)[_S * _INTER:].reshape(_INTER, 1)
    bsubf = (bsub * ssub + betsub)
    bias_z = jnp.sum(bsubf, axis=0)
    bias_full = jnp.concatenate(
        [bias_z.reshape(_S * _INTER), jnp.zeros((_INTER,), x.dtype)])
    bout = (bias_full * (gbn * _INV) + bbn).reshape(_C, 1, 1)
    return x2, x3, Wdt, bd, PAcat, Wblk, Mavg, WA, WB, bb, se, bout


@jax.jit
def kernel(x, PA, Wdown, bdown, gdown, betdown, Wsub, bsub, gsub, betsub,
           Wedge, gedge, betedge, gbn, bbn):
    args = _prep(x, PA, Wdown, bdown, gdown, betdown, Wsub, bsub, gsub,
                 betsub, Wedge, gedge, betedge, gbn, bbn)
    return _run(*args).reshape(_N, _C, _T, _V)
